# Initial kernel scaffold; baseline (speedup 1.0000x reference)
import jax
import jax.numpy as jnp
from jax.experimental import pallas as pl


def kernel(x, edge_index, edge_attr, batch, node_embs, edge_embs, conv_ws, proj_ws):
    raise NotImplementedError("write your pallas kernel here")


# speedup vs baseline: 3.1924x; 3.1924x over previous
import functools

import jax
import jax.numpy as jnp
import numpy as np
from jax import lax
from jax.experimental import pallas as pl
from jax.experimental.pallas import tpu as pltpu
from jax.experimental.pallas import tpu_sc as plsc

HIDDEN = 256
HALF = 128
N_NODES = 10000
N_EDGES = 160000
N_GRAPHS = 128
OUT_DIM = 768

NC = 2
NS = 16
LANES = 16

E_PER_SUB = N_EDGES // NS
EK = 80
N_CHUNKS = E_PER_SUB // EK
ROWS_PER_SUB = 624
ZROWS = 208
TAIL_ROWS = N_NODES - NS * ROWS_PER_SUB


def _sc_aggregate_body(h_hbm, e_hbm, src2_hbm, dst_hbm, out_hbm,
                       acc, idx_s, idx_d, hrows, erows, zbuf, gsem):
    cid = lax.axis_index("c")
    sid = lax.axis_index("s")

    def zero_row(r, carry):
        for c in range(HALF // LANES):
            zbuf[r, pl.ds(c * LANES, LANES)] = jnp.zeros((LANES,), jnp.float32)
        return carry
    lax.fori_loop(0, ZROWS, zero_row, 0)
    for z in range(ROWS_PER_SUB // ZROWS):
        pltpu.sync_copy(zbuf, acc.at[pl.ds(sid * ROWS_PER_SUB + z * ZROWS, ZROWS)])

    @pl.when(sid == 0)
    def _():
        pltpu.sync_copy(zbuf.at[pl.ds(0, TAIL_ROWS)],
                        acc.at[pl.ds(NS * ROWS_PER_SUB, TAIL_ROWS)])
    plsc.subcore_barrier()

    ebase = sid * E_PER_SUB

    def chunk(i, carry):
        b = ebase + i * EK
        pltpu.sync_copy(src2_hbm.at[pl.ds(cid * N_EDGES + b, EK)], idx_s)
        pltpu.sync_copy(dst_hbm.at[pl.ds(b, EK)], idx_d)
        gather = pltpu.async_copy(h_hbm.at[idx_s], hrows, gsem)
        pltpu.sync_copy(e_hbm.at[pl.ds(cid * N_EDGES + b, EK)], erows)
        gather.wait()

        def row(r, c2):
            for c in range(HALF // LANES):
                sl = pl.ds(c * LANES, LANES)
                erows[r, sl] = jnp.maximum(hrows[r, sl] + erows[r, sl], 0.0)
            return c2
        lax.fori_loop(0, EK, row, 0)

        pltpu.sync_copy(erows, acc.at[idx_d], add=True)
        return carry

    lax.fori_loop(0, N_CHUNKS, chunk, 0)
    plsc.subcore_barrier()

    for z in range(ROWS_PER_SUB // ZROWS):
        r0 = sid * ROWS_PER_SUB + z * ZROWS
        pltpu.sync_copy(acc.at[pl.ds(r0, ZROWS)],
                        out_hbm.at[pl.ds(cid * N_NODES + r0, ZROWS)])

    @pl.when(sid == 0)
    def _():
        t0 = NS * ROWS_PER_SUB
        pltpu.sync_copy(acc.at[pl.ds(t0, TAIL_ROWS)],
                        out_hbm.at[pl.ds(cid * N_NODES + t0, TAIL_ROWS)])


@functools.lru_cache(maxsize=None)
def _get_sc_aggregate():
    return pl.kernel(
        _sc_aggregate_body,
        out_type=jax.ShapeDtypeStruct((NC * N_NODES, HALF), jnp.float32),
        mesh=plsc.VectorSubcoreMesh(core_axis_name="c", subcore_axis_name="s",
                                    num_cores=NC, num_subcores=NS),
        scratch_types=[
            pltpu.VMEM_SHARED((N_NODES, HALF), jnp.float32),
            pltpu.VMEM((EK,), jnp.int32),
            pltpu.VMEM((EK,), jnp.int32),
            pltpu.VMEM((EK, HALF), jnp.float32),
            pltpu.VMEM((EK, HALF), jnp.float32),
            pltpu.VMEM((ZROWS, HALF), jnp.float32),
            pltpu.SemaphoreType.DMA,
        ],
    )


def _sc_aggregate(h2f, e2f, src2, dst):
    return _get_sc_aggregate()(h2f, e2f, src2, dst)


def _split_store(out_ref, m):
    out_ref[0] = m[:, :HALF]
    out_ref[1] = m[:, HALF:]


def _embed_body(n_fields, offs, idx_ref, tab_ref, out_ref):
    bn = idx_ref.shape[0]
    cats = tab_ref.shape[0]
    iota = lax.broadcasted_iota(jnp.int32, (bn, cats), 1)
    oh = jnp.zeros((bn, cats), jnp.float32)
    for j in range(n_fields):
        oh = oh + (idx_ref[:, j:j + 1] + offs[j] == iota).astype(jnp.float32)
    _split_store(out_ref, jnp.dot(oh, tab_ref[...],
                                  preferred_element_type=jnp.float32))


def _make_embed(idx, tables, cats_pad, bn):
    n, nf = idx.shape
    cards = [int(t.shape[0]) for t in tables]
    offs = np.concatenate([[0], np.cumsum(cards)[:-1]]).astype(np.int32).tolist()
    tab = jnp.concatenate(
        [jnp.concatenate(tables, axis=0),
         jnp.zeros((cats_pad - sum(cards), HIDDEN), jnp.float32)], axis=0)
    return pl.pallas_call(
        functools.partial(_embed_body, nf, offs),
        grid=(n // bn,),
        in_specs=[
            pl.BlockSpec((bn, nf), lambda i: (i, 0)),
            pl.BlockSpec((cats_pad, HIDDEN), lambda i: (0, 0)),
        ],
        out_specs=pl.BlockSpec((NC, bn, HALF), lambda i: (0, i, 0)),
        out_shape=jax.ShapeDtypeStruct((NC, n, HALF), jnp.float32),
    )(idx, tab)


def _mlp_body(h_ref, a_ref, w1_ref, b1_ref, w2_ref, b2_ref, out_ref):
    z = jnp.concatenate([h_ref[0] + a_ref[0], h_ref[1] + a_ref[1]], axis=1)
    z1 = jax.nn.relu(jnp.dot(z, w1_ref[...],
                             preferred_element_type=jnp.float32) + b1_ref[...])
    z2 = jnp.dot(z1, w2_ref[...], preferred_element_type=jnp.float32) + b2_ref[...]
    _split_store(out_ref, jax.nn.relu(z2))


def _mlp(h2, a2, w1, b1, w2, b2, bn):
    n = h2.shape[1]
    return pl.pallas_call(
        _mlp_body,
        grid=(n // bn,),
        in_specs=[
            pl.BlockSpec((NC, bn, HALF), lambda i: (0, i, 0)),
            pl.BlockSpec((NC, bn, HALF), lambda i: (0, i, 0)),
            pl.BlockSpec((HIDDEN, HIDDEN), lambda i: (0, 0)),
            pl.BlockSpec((1, HIDDEN), lambda i: (0, 0)),
            pl.BlockSpec((HIDDEN, HIDDEN), lambda i: (0, 0)),
            pl.BlockSpec((1, HIDDEN), lambda i: (0, 0)),
        ],
        out_specs=pl.BlockSpec((NC, bn, HALF), lambda i: (0, i, 0)),
        out_shape=jax.ShapeDtypeStruct((NC, n, HALF), jnp.float32),
    )(h2, a2, w1, b1.reshape(1, -1), w2, b2.reshape(1, -1))


def _final_body(nblk, h_ref, a_ref, w1_ref, b1_ref, w2_ref, b2_ref,
                batch_ref, p1_ref, pb1_ref, p2_ref, pb2_ref, out_ref, g_acc):
    i = pl.program_id(0)
    z = jnp.concatenate([h_ref[0] + a_ref[0], h_ref[1] + a_ref[1]], axis=1)
    z1 = jax.nn.relu(jnp.dot(z, w1_ref[...],
                             preferred_element_type=jnp.float32) + b1_ref[...])
    z2 = jnp.dot(z1, w2_ref[...], preferred_element_type=jnp.float32) + b2_ref[...]
    hn = jax.nn.relu(z2)
    bn = hn.shape[0]
    giota = lax.broadcasted_iota(jnp.int32, (N_GRAPHS, bn), 0)
    poh = (batch_ref[0] == giota).astype(jnp.float32)
    gp = jnp.dot(poh, hn, preferred_element_type=jnp.float32)

    @pl.when(i == 0)
    def _():
        g_acc[...] = gp

    @pl.when(i > 0)
    def _():
        g_acc[...] = g_acc[...] + gp

    @pl.when(i == nblk - 1)
    def _():
        g1 = jax.nn.relu(jnp.dot(g_acc[...], p1_ref[...],
                                 preferred_element_type=jnp.float32) + pb1_ref[...])
        g2 = jnp.dot(g1, p2_ref[...], preferred_element_type=jnp.float32) + pb2_ref[...]
        nrm = jnp.sqrt(jnp.sum(g2 * g2, axis=1, keepdims=True))
        out_ref[...] = g2 / jnp.maximum(nrm, 1e-12)


def _final(h2, a2, w1, b1, w2, b2, batch, p1, pb1, p2, pb2, bn):
    n = h2.shape[1]
    nblk = n // bn
    return pl.pallas_call(
        functools.partial(_final_body, nblk),
        grid=(nblk,),
        in_specs=[
            pl.BlockSpec((NC, bn, HALF), lambda i: (0, i, 0)),
            pl.BlockSpec((NC, bn, HALF), lambda i: (0, i, 0)),
            pl.BlockSpec((HIDDEN, HIDDEN), lambda i: (0, 0)),
            pl.BlockSpec((1, HIDDEN), lambda i: (0, 0)),
            pl.BlockSpec((HIDDEN, HIDDEN), lambda i: (0, 0)),
            pl.BlockSpec((1, HIDDEN), lambda i: (0, 0)),
            pl.BlockSpec((1, 1, bn), lambda i: (i, 0, 0)),
            pl.BlockSpec((HIDDEN, HIDDEN), lambda i: (0, 0)),
            pl.BlockSpec((1, HIDDEN), lambda i: (0, 0)),
            pl.BlockSpec((HIDDEN, OUT_DIM), lambda i: (0, 0)),
            pl.BlockSpec((1, OUT_DIM), lambda i: (0, 0)),
        ],
        out_specs=pl.BlockSpec((N_GRAPHS, OUT_DIM), lambda i: (0, 0)),
        out_shape=jax.ShapeDtypeStruct((N_GRAPHS, OUT_DIM), jnp.float32),
        scratch_shapes=[pltpu.VMEM((N_GRAPHS, HIDDEN), jnp.float32)],
    )(h2, a2, w1, b1.reshape(1, -1), w2, b2.reshape(1, -1),
      batch.reshape(nblk, 1, bn), p1, pb1.reshape(1, -1), p2, pb2.reshape(1, -1))


def kernel(x, edge_index, edge_attr, batch, node_embs, edge_embs, conv_ws, proj_ws):
    src = edge_index[0]
    dst = edge_index[1]
    src2 = jnp.concatenate([src, src + N_NODES])

    h2 = _make_embed(x, node_embs, 256, 1000)
    e2 = _make_embed(edge_attr, edge_embs, 128, 2000)
    e2f = e2.reshape(NC * N_EDGES, HALF)

    for li, (w1, b1, w2, b2) in enumerate(conv_ws):
        a2 = _sc_aggregate(h2.reshape(NC * N_NODES, HALF), e2f, src2, dst)
        a2 = a2.reshape(NC, N_NODES, HALF)
        if li < len(conv_ws) - 1:
            h2 = _mlp(h2, a2, w1, b1, w2, b2, 1000)
        else:
            p1, pb1, p2, pb2 = proj_ws
            return _final(h2, a2, w1, b1, w2, b2, batch, p1, pb1, p2, pb2, 1000)


# speedup vs baseline: 6.1135x; 1.9150x over previous
import functools

import jax
import jax.numpy as jnp
import numpy as np
from jax import lax
from jax.experimental import pallas as pl
from jax.experimental.pallas import tpu as pltpu
from jax.experimental.pallas import tpu_sc as plsc

HIDDEN = 256
HALF = 128
N_NODES = 10000
N_EDGES = 160000
N_GRAPHS = 128
OUT_DIM = 768

NC = 2
NS = 16
LANES = 16

E_PER_SUB = N_EDGES // NS
EK = 80
N_CHUNKS = E_PER_SUB // EK
ROWS_PER_SUB = 624
ZROWS = 48
TAIL_ROWS = N_NODES - NS * ROWS_PER_SUB
CHUNKS_PER_CORE = N_EDGES // EK


def _sc_aggregate_body(h_hbm, e_hbm, sd_hbm, out_hbm,
                       acc, ix0, ix1, ix2, hrows0, hrows1,
                       erows0, erows1, zbuf, gs0, gs1, es0, es1, is0, is1, is2):
    cid = lax.axis_index("c")
    sid = lax.axis_index("s")

    def zero_row(r, carry):
        for c in range(HALF // LANES):
            zbuf[r, pl.ds(c * LANES, LANES)] = jnp.zeros((LANES,), jnp.float32)
        return carry
    lax.fori_loop(0, ZROWS, zero_row, 0)
    for z in range(ROWS_PER_SUB // ZROWS):
        pltpu.sync_copy(zbuf, acc.at[pl.ds(sid * ROWS_PER_SUB + z * ZROWS, ZROWS)])

    @pl.when(sid == 0)
    def _():
        pltpu.sync_copy(zbuf.at[pl.ds(0, TAIL_ROWS)],
                        acc.at[pl.ds(NS * ROWS_PER_SUB, TAIL_ROWS)])
    plsc.subcore_barrier()

    cbase = cid * CHUNKS_PER_CORE + sid * N_CHUNKS
    ixs = ((ix0, is0), (ix1, is1), (ix2, is2))
    dbufs = ((hrows0, erows0, gs0, es0), (hrows1, erows1, gs1, es1))

    def ix_cp(i, b3):
        ix, sem = ixs[b3]
        return pltpu.make_async_copy(sd_hbm.at[cbase + i], ix, sem)

    def gather_cp(i, b2, b3):
        return pltpu.make_async_copy(
            h_hbm.at[ixs[b3][0].at[0]], dbufs[b2][0], dbufs[b2][2])

    def e_cp(i, b2):
        return pltpu.make_async_copy(
            e_hbm.at[pl.ds(cid * N_EDGES + (sid * N_CHUNKS + i) * EK, EK)],
            dbufs[b2][1], dbufs[b2][3])

    def compute_scatter(i, b2, b3):
        hbuf, ebuf, gs, es = dbufs[b2]
        gather_cp(i, b2, b3).wait()
        e_cp(i, b2).wait()

        def row(r, c2):
            for c in range(HALF // LANES):
                sl = pl.ds(c * LANES, LANES)
                ebuf[r, sl] = jnp.maximum(hbuf[r, sl] + ebuf[r, sl], 0.0)
            return c2
        lax.fori_loop(0, EK, row, 0)
        pltpu.sync_copy(ebuf, acc.at[ixs[b3][0].at[1]], add=True)

    def step(i, b2, b3, issue_next, issue_next2):
        if issue_next:
            ix_cp(i + 1, (b3 + 1) % 3).wait()
            gather_cp(i + 1, 1 - b2, (b3 + 1) % 3).start()
            e_cp(i + 1, 1 - b2).start()
        if issue_next2:
            ix_cp(i + 2, (b3 + 2) % 3).start()
        compute_scatter(i, b2, b3)

    ix_cp(0, 0).start()
    ix_cp(0, 0).wait()
    gather_cp(0, 0, 0).start()
    e_cp(0, 0).start()
    ix_cp(1, 1).start()

    PERIOD = 6
    MAIN = (N_CHUNKS - 1) // PERIOD * PERIOD

    def six(g, carry):
        for b in range(PERIOD):
            i6 = g * PERIOD + b
            step(i6, b % 2, b % 3, True, True)
        return carry

    lax.fori_loop(0, MAIN // PERIOD, six, 0)
    for i in range(MAIN, N_CHUNKS):
        step(i, i % 2, i % 3, i + 1 < N_CHUNKS, i + 2 < N_CHUNKS)
    plsc.subcore_barrier()

    for z in range(ROWS_PER_SUB // ZROWS):
        r0 = sid * ROWS_PER_SUB + z * ZROWS
        pltpu.sync_copy(acc.at[pl.ds(r0, ZROWS)],
                        out_hbm.at[pl.ds(cid * N_NODES + r0, ZROWS)])

    @pl.when(sid == 0)
    def _():
        t0 = NS * ROWS_PER_SUB
        pltpu.sync_copy(acc.at[pl.ds(t0, TAIL_ROWS)],
                        out_hbm.at[pl.ds(cid * N_NODES + t0, TAIL_ROWS)])


@functools.lru_cache(maxsize=None)
def _get_sc_aggregate():
    return pl.kernel(
        _sc_aggregate_body,
        out_type=jax.ShapeDtypeStruct((NC * N_NODES, HALF), jnp.float32),
        mesh=plsc.VectorSubcoreMesh(core_axis_name="c", subcore_axis_name="s",
                                    num_cores=NC, num_subcores=NS),
        scratch_types=[
            pltpu.VMEM_SHARED((N_NODES, HALF), jnp.float32),
            pltpu.VMEM((2, EK), jnp.int32),
            pltpu.VMEM((2, EK), jnp.int32),
            pltpu.VMEM((2, EK), jnp.int32),
            pltpu.VMEM((EK, HALF), jnp.float32),
            pltpu.VMEM((EK, HALF), jnp.float32),
            pltpu.VMEM((EK, HALF), jnp.float32),
            pltpu.VMEM((EK, HALF), jnp.float32),
            pltpu.VMEM((ZROWS, HALF), jnp.float32),
            pltpu.SemaphoreType.DMA,
            pltpu.SemaphoreType.DMA,
            pltpu.SemaphoreType.DMA,
            pltpu.SemaphoreType.DMA,
            pltpu.SemaphoreType.DMA,
            pltpu.SemaphoreType.DMA,
            pltpu.SemaphoreType.DMA,
        ],
    )


def _sc_aggregate(h2f, e2f, sd):
    return _get_sc_aggregate()(h2f, e2f, sd)


def _split_store(out_ref, m):
    out_ref[0] = m[:, :HALF]
    out_ref[1] = m[:, HALF:]


def _embed_body(n_fields, offs, idx_ref, tab_ref, out_ref):
    bn = idx_ref.shape[0]
    cats = tab_ref.shape[0]
    iota = lax.broadcasted_iota(jnp.int32, (bn, cats), 1)
    oh = jnp.zeros((bn, cats), jnp.float32)
    for j in range(n_fields):
        oh = oh + (idx_ref[:, j:j + 1] + offs[j] == iota).astype(jnp.float32)
    _split_store(out_ref, jnp.dot(oh, tab_ref[...],
                                  preferred_element_type=jnp.float32))


def _make_embed(idx, tables, cats_pad, bn):
    n, nf = idx.shape
    cards = [int(t.shape[0]) for t in tables]
    offs = np.concatenate([[0], np.cumsum(cards)[:-1]]).astype(np.int32).tolist()
    tab = jnp.concatenate(
        [jnp.concatenate(tables, axis=0),
         jnp.zeros((cats_pad - sum(cards), HIDDEN), jnp.float32)], axis=0)
    return pl.pallas_call(
        functools.partial(_embed_body, nf, offs),
        grid=(n // bn,),
        in_specs=[
            pl.BlockSpec((bn, nf), lambda i: (i, 0)),
            pl.BlockSpec((cats_pad, HIDDEN), lambda i: (0, 0)),
        ],
        out_specs=pl.BlockSpec((NC, bn, HALF), lambda i: (0, i, 0)),
        out_shape=jax.ShapeDtypeStruct((NC, n, HALF), jnp.float32),
    )(idx, tab)


def _mlp_body(h_ref, a_ref, w1_ref, b1_ref, w2_ref, b2_ref, out_ref):
    z = jnp.concatenate([h_ref[0] + a_ref[0], h_ref[1] + a_ref[1]], axis=1)
    z1 = jax.nn.relu(jnp.dot(z, w1_ref[...],
                             preferred_element_type=jnp.float32) + b1_ref[...])
    z2 = jnp.dot(z1, w2_ref[...], preferred_element_type=jnp.float32) + b2_ref[...]
    _split_store(out_ref, jax.nn.relu(z2))


def _mlp(h2, a2, w1, b1, w2, b2, bn):
    n = h2.shape[1]
    return pl.pallas_call(
        _mlp_body,
        grid=(n // bn,),
        in_specs=[
            pl.BlockSpec((NC, bn, HALF), lambda i: (0, i, 0)),
            pl.BlockSpec((NC, bn, HALF), lambda i: (0, i, 0)),
            pl.BlockSpec((HIDDEN, HIDDEN), lambda i: (0, 0)),
            pl.BlockSpec((1, HIDDEN), lambda i: (0, 0)),
            pl.BlockSpec((HIDDEN, HIDDEN), lambda i: (0, 0)),
            pl.BlockSpec((1, HIDDEN), lambda i: (0, 0)),
        ],
        out_specs=pl.BlockSpec((NC, bn, HALF), lambda i: (0, i, 0)),
        out_shape=jax.ShapeDtypeStruct((NC, n, HALF), jnp.float32),
    )(h2, a2, w1, b1.reshape(1, -1), w2, b2.reshape(1, -1))


def _final_body(nblk, h_ref, a_ref, w1_ref, b1_ref, w2_ref, b2_ref,
                batch_ref, p1_ref, pb1_ref, p2_ref, pb2_ref, out_ref, g_acc):
    i = pl.program_id(0)
    z = jnp.concatenate([h_ref[0] + a_ref[0], h_ref[1] + a_ref[1]], axis=1)
    z1 = jax.nn.relu(jnp.dot(z, w1_ref[...],
                             preferred_element_type=jnp.float32) + b1_ref[...])
    z2 = jnp.dot(z1, w2_ref[...], preferred_element_type=jnp.float32) + b2_ref[...]
    hn = jax.nn.relu(z2)
    bn = hn.shape[0]
    giota = lax.broadcasted_iota(jnp.int32, (N_GRAPHS, bn), 0)
    poh = (batch_ref[0] == giota).astype(jnp.float32)
    gp = jnp.dot(poh, hn, preferred_element_type=jnp.float32)

    @pl.when(i == 0)
    def _():
        g_acc[...] = gp

    @pl.when(i > 0)
    def _():
        g_acc[...] = g_acc[...] + gp

    @pl.when(i == nblk - 1)
    def _():
        g1 = jax.nn.relu(jnp.dot(g_acc[...], p1_ref[...],
                                 preferred_element_type=jnp.float32) + pb1_ref[...])
        g2 = jnp.dot(g1, p2_ref[...], preferred_element_type=jnp.float32) + pb2_ref[...]
        nrm = jnp.sqrt(jnp.sum(g2 * g2, axis=1, keepdims=True))
        out_ref[...] = g2 / jnp.maximum(nrm, 1e-12)


def _final(h2, a2, w1, b1, w2, b2, batch, p1, pb1, p2, pb2, bn):
    n = h2.shape[1]
    nblk = n // bn
    return pl.pallas_call(
        functools.partial(_final_body, nblk),
        grid=(nblk,),
        in_specs=[
            pl.BlockSpec((NC, bn, HALF), lambda i: (0, i, 0)),
            pl.BlockSpec((NC, bn, HALF), lambda i: (0, i, 0)),
            pl.BlockSpec((HIDDEN, HIDDEN), lambda i: (0, 0)),
            pl.BlockSpec((1, HIDDEN), lambda i: (0, 0)),
            pl.BlockSpec((HIDDEN, HIDDEN), lambda i: (0, 0)),
            pl.BlockSpec((1, HIDDEN), lambda i: (0, 0)),
            pl.BlockSpec((1, 1, bn), lambda i: (i, 0, 0)),
            pl.BlockSpec((HIDDEN, HIDDEN), lambda i: (0, 0)),
            pl.BlockSpec((1, HIDDEN), lambda i: (0, 0)),
            pl.BlockSpec((HIDDEN, OUT_DIM), lambda i: (0, 0)),
            pl.BlockSpec((1, OUT_DIM), lambda i: (0, 0)),
        ],
        out_specs=pl.BlockSpec((N_GRAPHS, OUT_DIM), lambda i: (0, 0)),
        out_shape=jax.ShapeDtypeStruct((N_GRAPHS, OUT_DIM), jnp.float32),
        scratch_shapes=[pltpu.VMEM((N_GRAPHS, HIDDEN), jnp.float32)],
    )(h2, a2, w1, b1.reshape(1, -1), w2, b2.reshape(1, -1),
      batch.reshape(nblk, 1, bn), p1, pb1.reshape(1, -1), p2, pb2.reshape(1, -1))


def kernel(x, edge_index, edge_attr, batch, node_embs, edge_embs, conv_ws, proj_ws):
    src = edge_index[0]
    dst = edge_index[1]
    src2r = jnp.stack([src, src + N_NODES]).reshape(NC, CHUNKS_PER_CORE, EK)
    dstr = jnp.broadcast_to(dst.reshape(CHUNKS_PER_CORE, EK),
                            (NC, CHUNKS_PER_CORE, EK))
    sd = jnp.stack([src2r, dstr], axis=2).reshape(NC * CHUNKS_PER_CORE, 2, EK)

    h2 = _make_embed(x, node_embs, 256, 1000)
    e2 = _make_embed(edge_attr, edge_embs, 128, 2000)
    e2f = e2.reshape(NC * N_EDGES, HALF)

    for li, (w1, b1, w2, b2) in enumerate(conv_ws):
        a2 = _sc_aggregate(h2.reshape(NC * N_NODES, HALF), e2f, sd)
        a2 = a2.reshape(NC, N_NODES, HALF)
        if li < len(conv_ws) - 1:
            h2 = _mlp(h2, a2, w1, b1, w2, b2, 1000)
        else:
            p1, pb1, p2, pb2 = proj_ws
            return _final(h2, a2, w1, b1, w2, b2, batch, p1, pb1, p2, pb2, 1000)


# speedup vs baseline: 6.1463x; 1.0054x over previous
import functools

import jax
import jax.numpy as jnp
import numpy as np
from jax import lax
from jax.experimental import pallas as pl
from jax.experimental.pallas import tpu as pltpu
from jax.experimental.pallas import tpu_sc as plsc

HIDDEN = 256
HALF = 128
N_NODES = 10000
N_EDGES = 160000
N_GRAPHS = 128
OUT_DIM = 768

NC = 2
NS = 16
LANES = 16

E_PER_SUB = N_EDGES // NS
EK = 80
N_CHUNKS = E_PER_SUB // EK
ROWS_PER_SUB = 624
ZROWS = 48
TAIL_ROWS = N_NODES - NS * ROWS_PER_SUB
CHUNKS_PER_CORE = N_EDGES // EK


def _sc_aggregate_body(h_hbm, e_hbm, sd_hbm, out_hbm,
                       acc, ix0, ix1, ix2, hrows0, hrows1,
                       erows0, erows1, zbuf, gs0, gs1, es0, es1, is0, is1, is2):
    cid = lax.axis_index("c")
    sid = lax.axis_index("s")

    def zero_row(r, carry):
        for c in range(HALF // LANES):
            zbuf[r, pl.ds(c * LANES, LANES)] = jnp.zeros((LANES,), jnp.float32)
        return carry
    lax.fori_loop(0, ZROWS, zero_row, 0)
    for z in range(ROWS_PER_SUB // ZROWS):
        pltpu.sync_copy(zbuf, acc.at[pl.ds(sid * ROWS_PER_SUB + z * ZROWS, ZROWS)])

    @pl.when(sid == 0)
    def _():
        pltpu.sync_copy(zbuf.at[pl.ds(0, TAIL_ROWS)],
                        acc.at[pl.ds(NS * ROWS_PER_SUB, TAIL_ROWS)])
    plsc.subcore_barrier()

    cbase = cid * CHUNKS_PER_CORE + sid * N_CHUNKS
    ixs = ((ix0, is0), (ix1, is1), (ix2, is2))
    dbufs = ((hrows0, erows0, gs0, es0), (hrows1, erows1, gs1, es1))

    def ix_cp(i, b3):
        ix, sem = ixs[b3]
        return pltpu.make_async_copy(sd_hbm.at[cbase + i], ix, sem)

    def gather_cp(i, b2, b3):
        return pltpu.make_async_copy(
            h_hbm.at[ixs[b3][0].at[0]], dbufs[b2][0], dbufs[b2][2])

    def e_cp(i, b2):
        return pltpu.make_async_copy(
            e_hbm.at[pl.ds(cid * N_EDGES + (sid * N_CHUNKS + i) * EK, EK)],
            dbufs[b2][1], dbufs[b2][3])

    def compute_scatter(i, b2, b3):
        hbuf, ebuf, gs, es = dbufs[b2]
        gather_cp(i, b2, b3).wait()
        e_cp(i, b2).wait()

        def row2(r, c2):
            for u in range(2):
                for c in range(HALF // LANES):
                    sl = pl.ds(c * LANES, LANES)
                    ebuf[2 * r + u, sl] = jnp.maximum(
                        hbuf[2 * r + u, sl] + ebuf[2 * r + u, sl], 0.0)
            return c2
        lax.fori_loop(0, EK // 2, row2, 0)
        pltpu.sync_copy(ebuf, acc.at[ixs[b3][0].at[1]], add=True)

    def step(i, b2, b3, issue_next, issue_next2):
        if issue_next:
            ix_cp(i + 1, (b3 + 1) % 3).wait()
            gather_cp(i + 1, 1 - b2, (b3 + 1) % 3).start()
            e_cp(i + 1, 1 - b2).start()
        if issue_next2:
            ix_cp(i + 2, (b3 + 2) % 3).start()
        compute_scatter(i, b2, b3)


    ix_cp(0, 0).start()
    ix_cp(0, 0).wait()
    gather_cp(0, 0, 0).start()
    e_cp(0, 0).start()
    ix_cp(1, 1).start()

    PERIOD = 6
    MAIN = (N_CHUNKS - 1) // PERIOD * PERIOD

    def six(g, carry):
        for b in range(PERIOD):
            i6 = g * PERIOD + b
            step(i6, b % 2, b % 3, True, True)
        return carry

    lax.fori_loop(0, MAIN // PERIOD, six, 0)
    for i in range(MAIN, N_CHUNKS):
        step(i, i % 2, i % 3, i + 1 < N_CHUNKS, i + 2 < N_CHUNKS)
    plsc.subcore_barrier()

    for z in range(ROWS_PER_SUB // ZROWS):
        r0 = sid * ROWS_PER_SUB + z * ZROWS
        pltpu.sync_copy(acc.at[pl.ds(r0, ZROWS)],
                        out_hbm.at[pl.ds(cid * N_NODES + r0, ZROWS)])

    @pl.when(sid == 0)
    def _():
        t0 = NS * ROWS_PER_SUB
        pltpu.sync_copy(acc.at[pl.ds(t0, TAIL_ROWS)],
                        out_hbm.at[pl.ds(cid * N_NODES + t0, TAIL_ROWS)])


@functools.lru_cache(maxsize=None)
def _get_sc_aggregate():
    return pl.kernel(
        _sc_aggregate_body,
        out_type=jax.ShapeDtypeStruct((NC * N_NODES, HALF), jnp.float32),
        mesh=plsc.VectorSubcoreMesh(core_axis_name="c", subcore_axis_name="s",
                                    num_cores=NC, num_subcores=NS),
        scratch_types=[
            pltpu.VMEM_SHARED((N_NODES, HALF), jnp.float32),
            pltpu.VMEM((2, EK), jnp.int32),
            pltpu.VMEM((2, EK), jnp.int32),
            pltpu.VMEM((2, EK), jnp.int32),
            pltpu.VMEM((EK, HALF), jnp.float32),
            pltpu.VMEM((EK, HALF), jnp.float32),
            pltpu.VMEM((EK, HALF), jnp.float32),
            pltpu.VMEM((EK, HALF), jnp.float32),
            pltpu.VMEM((ZROWS, HALF), jnp.float32),
            pltpu.SemaphoreType.DMA,
            pltpu.SemaphoreType.DMA,
            pltpu.SemaphoreType.DMA,
            pltpu.SemaphoreType.DMA,
            pltpu.SemaphoreType.DMA,
            pltpu.SemaphoreType.DMA,
            pltpu.SemaphoreType.DMA,
        ],
    )


def _sc_aggregate(h2f, e2f, sd):
    return _get_sc_aggregate()(h2f, e2f, sd)


def _split_store(out_ref, m):
    out_ref[0] = m[:, :HALF]
    out_ref[1] = m[:, HALF:]


def _embed_body(n_fields, offs, idx_ref, tab_ref, out_ref):
    bn = idx_ref.shape[0]
    cats = tab_ref.shape[0]
    iota = lax.broadcasted_iota(jnp.int32, (bn, cats), 1)
    oh = jnp.zeros((bn, cats), jnp.float32)
    for j in range(n_fields):
        oh = oh + (idx_ref[:, j:j + 1] + offs[j] == iota).astype(jnp.float32)
    _split_store(out_ref, jnp.dot(oh, tab_ref[...],
                                  preferred_element_type=jnp.float32))


def _make_embed(idx, tables, cats_pad, bn):
    n, nf = idx.shape
    cards = [int(t.shape[0]) for t in tables]
    offs = np.concatenate([[0], np.cumsum(cards)[:-1]]).astype(np.int32).tolist()
    tab = jnp.concatenate(
        [jnp.concatenate(tables, axis=0),
         jnp.zeros((cats_pad - sum(cards), HIDDEN), jnp.float32)], axis=0)
    return pl.pallas_call(
        functools.partial(_embed_body, nf, offs),
        grid=(n // bn,),
        in_specs=[
            pl.BlockSpec((bn, nf), lambda i: (i, 0)),
            pl.BlockSpec((cats_pad, HIDDEN), lambda i: (0, 0)),
        ],
        out_specs=pl.BlockSpec((NC, bn, HALF), lambda i: (0, i, 0)),
        out_shape=jax.ShapeDtypeStruct((NC, n, HALF), jnp.float32),
    )(idx, tab)


def _mlp_body(h_ref, a_ref, w1_ref, b1_ref, w2_ref, b2_ref, out_ref):
    z = jnp.concatenate([h_ref[0] + a_ref[0], h_ref[1] + a_ref[1]], axis=1)
    z1 = jax.nn.relu(jnp.dot(z, w1_ref[...],
                             preferred_element_type=jnp.float32) + b1_ref[...])
    z2 = jnp.dot(z1, w2_ref[...], preferred_element_type=jnp.float32) + b2_ref[...]
    _split_store(out_ref, jax.nn.relu(z2))


def _mlp(h2, a2, w1, b1, w2, b2, bn):
    n = h2.shape[1]
    return pl.pallas_call(
        _mlp_body,
        grid=(n // bn,),
        in_specs=[
            pl.BlockSpec((NC, bn, HALF), lambda i: (0, i, 0)),
            pl.BlockSpec((NC, bn, HALF), lambda i: (0, i, 0)),
            pl.BlockSpec((HIDDEN, HIDDEN), lambda i: (0, 0)),
            pl.BlockSpec((1, HIDDEN), lambda i: (0, 0)),
            pl.BlockSpec((HIDDEN, HIDDEN), lambda i: (0, 0)),
            pl.BlockSpec((1, HIDDEN), lambda i: (0, 0)),
        ],
        out_specs=pl.BlockSpec((NC, bn, HALF), lambda i: (0, i, 0)),
        out_shape=jax.ShapeDtypeStruct((NC, n, HALF), jnp.float32),
    )(h2, a2, w1, b1.reshape(1, -1), w2, b2.reshape(1, -1))


def _final_body(nblk, h_ref, a_ref, w1_ref, b1_ref, w2_ref, b2_ref,
                batch_ref, p1_ref, pb1_ref, p2_ref, pb2_ref, out_ref, g_acc):
    i = pl.program_id(0)
    z = jnp.concatenate([h_ref[0] + a_ref[0], h_ref[1] + a_ref[1]], axis=1)
    z1 = jax.nn.relu(jnp.dot(z, w1_ref[...],
                             preferred_element_type=jnp.float32) + b1_ref[...])
    z2 = jnp.dot(z1, w2_ref[...], preferred_element_type=jnp.float32) + b2_ref[...]
    hn = jax.nn.relu(z2)
    bn = hn.shape[0]
    giota = lax.broadcasted_iota(jnp.int32, (N_GRAPHS, bn), 0)
    poh = (batch_ref[0] == giota).astype(jnp.float32)
    gp = jnp.dot(poh, hn, preferred_element_type=jnp.float32)

    @pl.when(i == 0)
    def _():
        g_acc[...] = gp

    @pl.when(i > 0)
    def _():
        g_acc[...] = g_acc[...] + gp

    @pl.when(i == nblk - 1)
    def _():
        g1 = jax.nn.relu(jnp.dot(g_acc[...], p1_ref[...],
                                 preferred_element_type=jnp.float32) + pb1_ref[...])
        g2 = jnp.dot(g1, p2_ref[...], preferred_element_type=jnp.float32) + pb2_ref[...]
        nrm = jnp.sqrt(jnp.sum(g2 * g2, axis=1, keepdims=True))
        out_ref[...] = g2 / jnp.maximum(nrm, 1e-12)


def _final(h2, a2, w1, b1, w2, b2, batch, p1, pb1, p2, pb2, bn):
    n = h2.shape[1]
    nblk = n // bn
    return pl.pallas_call(
        functools.partial(_final_body, nblk),
        grid=(nblk,),
        in_specs=[
            pl.BlockSpec((NC, bn, HALF), lambda i: (0, i, 0)),
            pl.BlockSpec((NC, bn, HALF), lambda i: (0, i, 0)),
            pl.BlockSpec((HIDDEN, HIDDEN), lambda i: (0, 0)),
            pl.BlockSpec((1, HIDDEN), lambda i: (0, 0)),
            pl.BlockSpec((HIDDEN, HIDDEN), lambda i: (0, 0)),
            pl.BlockSpec((1, HIDDEN), lambda i: (0, 0)),
            pl.BlockSpec((1, 1, bn), lambda i: (i, 0, 0)),
            pl.BlockSpec((HIDDEN, HIDDEN), lambda i: (0, 0)),
            pl.BlockSpec((1, HIDDEN), lambda i: (0, 0)),
            pl.BlockSpec((HIDDEN, OUT_DIM), lambda i: (0, 0)),
            pl.BlockSpec((1, OUT_DIM), lambda i: (0, 0)),
        ],
        out_specs=pl.BlockSpec((N_GRAPHS, OUT_DIM), lambda i: (0, 0)),
        out_shape=jax.ShapeDtypeStruct((N_GRAPHS, OUT_DIM), jnp.float32),
        scratch_shapes=[pltpu.VMEM((N_GRAPHS, HIDDEN), jnp.float32)],
    )(h2, a2, w1, b1.reshape(1, -1), w2, b2.reshape(1, -1),
      batch.reshape(nblk, 1, bn), p1, pb1.reshape(1, -1), p2, pb2.reshape(1, -1))


def kernel(x, edge_index, edge_attr, batch, node_embs, edge_embs, conv_ws, proj_ws):
    src = edge_index[0]
    dst = edge_index[1]
    src2r = jnp.stack([src, src + N_NODES]).reshape(NC, CHUNKS_PER_CORE, EK)
    dstr = jnp.broadcast_to(dst.reshape(CHUNKS_PER_CORE, EK),
                            (NC, CHUNKS_PER_CORE, EK))
    sd = jnp.stack([src2r, dstr], axis=2).reshape(NC * CHUNKS_PER_CORE, 2, EK)

    h2 = _make_embed(x, node_embs, 256, 1000)
    e2 = _make_embed(edge_attr, edge_embs, 128, 2000)
    e2f = e2.reshape(NC * N_EDGES, HALF)

    for li, (w1, b1, w2, b2) in enumerate(conv_ws):
        a2 = _sc_aggregate(h2.reshape(NC * N_NODES, HALF), e2f, sd)
        a2 = a2.reshape(NC, N_NODES, HALF)
        if li < len(conv_ws) - 1:
            h2 = _mlp(h2, a2, w1, b1, w2, b2, 1000)
        else:
            p1, pb1, p2, pb2 = proj_ws
            return _final(h2, a2, w1, b1, w2, b2, batch, p1, pb1, p2, pb2, 1000)
